# baseline (speedup 1.0000x reference)
import jax
import jax.numpy as jnp
from jax import lax
from jax.experimental import pallas as pl
from jax.experimental.pallas import tpu as pltpu

_B, _L, _E = 16384, 50, 64
_S = 20
_LE = _L * _E
_BB = 512


def _fused_body(idx_ref, table_ref, pos_ref, item_ref, embed_ref, out_ref):
    idx = idx_ref[...]
    table = table_ref[...]
    s_iota = lax.broadcasted_iota(jnp.int32, (1, _S), 1)
    sb = jnp.sum(jnp.where(idx == s_iota, table, 0.0), axis=1, keepdims=True)
    out_ref[...] = embed_ref[...] + sb + (pos_ref[...] + item_ref[...])


def kernel(session_embed, session_index, session_bias, position_bias, item_bias):
    embed2d = session_embed.reshape(_B, _LE)
    idx2d = session_index.astype(jnp.int32).reshape(_B, 1)
    table = session_bias.reshape(1, _S)
    pos2d = jnp.broadcast_to(position_bias, (1, _L, _E)).reshape(1, _LE)
    item2d = jnp.broadcast_to(item_bias, (1, _L, _E)).reshape(1, _LE)

    grid = (_B // _BB,)
    out2d = pl.pallas_call(
        _fused_body,
        grid=grid,
        in_specs=[
            pl.BlockSpec((_BB, 1), lambda i: (i, 0)),
            pl.BlockSpec((1, _S), lambda i: (0, 0)),
            pl.BlockSpec((1, _LE), lambda i: (0, 0)),
            pl.BlockSpec((1, _LE), lambda i: (0, 0)),
            pl.BlockSpec((_BB, _LE), lambda i: (i, 0)),
        ],
        out_specs=pl.BlockSpec((_BB, _LE), lambda i: (i, 0)),
        out_shape=jax.ShapeDtypeStruct((_B, _LE), jnp.float32),
        compiler_params=pltpu.CompilerParams(
            dimension_semantics=("arbitrary",),
        ),
    )(idx2d, table, pos2d, item2d, embed2d)
    return out2d.reshape(_B, _L, _E)
